# baseline (speedup 1.0000x reference)
import functools

import jax
import jax.numpy as jnp
from jax import lax
from jax.experimental import pallas as pl
from jax.experimental.pallas import tpu as pltpu
from jax.experimental.pallas import tpu_sc as plsc

_ITERATIONS = 5
_STAGES = _ITERATIONS + 1
_LR_POS = 0.45
_LR_SIZE = 0.4
_L = 16
_KQ = 4


@functools.lru_cache(maxsize=None)
def _make_sc_call(B, N, G):
    assert B == 2, "kernel specialized to B == 2"
    assert N % _L == 0 and G % _L == 0

    info = plsc.get_sparse_core_info()
    NW = info.num_cores * info.num_subcores
    WPB = NW // B
    nbg = N // _L
    base_loc = nbg // WPB
    n_rem = nbg - base_loc * WPB
    n_quads = -(-(base_loc + 1) // _KQ)
    assert n_quads % 2 == 0, "quad ring expects an even quad count"
    assert base_loc >= _KQ

    S = _STAGES
    GSLAB = S * _L * G
    GRSLAB = S * _L * 4
    SLAB = _KQ * GSLAB
    RSLAB = _KQ * GRSLAB

    mesh = plsc.VectorSubcoreMesh(core_axis_name="c", subcore_axis_name="s")

    def body(boxes_in, gt_in, res_out, sims_out,
             gt_v, area_v, inbox, slab0, slab1, rslab0, rslab1, sem0, sem1):
        wid = lax.axis_index("s") * info.num_cores + lax.axis_index("c")
        b = (wid >= WPB).astype(jnp.int32)
        lid = wid - b * WPB
        loc = base_loc + (lid < n_rem).astype(jnp.int32)
        start = base_loc * lid + jnp.minimum(lid, n_rem)

        pltpu.sync_copy(gt_in, gt_v)

        iota = lax.iota(jnp.int32, _L)
        col4 = iota * 4
        colG = iota * G

        for i in range(B * G // _L):
            ibase = col4 + (i * _L * 4)
            x1 = plsc.load_gather(gt_v, [ibase])
            y1 = plsc.load_gather(gt_v, [ibase + 1])
            x2 = plsc.load_gather(gt_v, [ibase + 2])
            y2 = plsc.load_gather(gt_v, [ibase + 3])
            area_v[pl.ds(i * _L, _L)] = (x2 - x1) * (y2 - y1)

        gofs = b * (G * 4)
        aofs = b * G

        def drain(slab, rslab, sem):
            pltpu.make_async_copy(sims_out.at[pl.ds(0, SLAB)], slab, sem).wait()
            pltpu.make_async_copy(res_out.at[pl.ds(0, RSLAB)], rslab, sem).wait()

        def process_quad(q, slab, rslab, sem):
            jj0 = start + jnp.minimum(q * _KQ, loc - _KQ)
            n0q = jj0 * _L
            cur = []
            pltpu.sync_copy(
                boxes_in.at[pl.ds((b * N + n0q) * 4, _KQ * _L * 4)], inbox)
            for k in range(_KQ):
                kb = col4 + k * (_L * 4)
                cur.append((plsc.load_gather(inbox, [kb]),
                            plsc.load_gather(inbox, [kb + 1]),
                            plsc.load_gather(inbox, [kb + 2]),
                            plsc.load_gather(inbox, [kb + 3])))
            for s in range(S):
                areas, sbases = [], []
                for k in range(_KQ):
                    px1, py1, px2, py2 = cur[k]
                    rbase = col4 + (s * _KQ + k) * (_L * 4)
                    plsc.store_scatter(rslab, [rbase], px1)
                    plsc.store_scatter(rslab, [rbase + 1], py1)
                    plsc.store_scatter(rslab, [rbase + 2], px2)
                    plsc.store_scatter(rslab, [rbase + 3], py2)
                    areas.append((px2 - px1) * (py2 - py1))
                    sbases.append(colG + (s * _KQ + k) * (_L * G))

                def gt_body(g, carry, _cur=tuple(cur), _areas=tuple(areas),
                            _sbases=tuple(sbases), _slab=slab):
                    bvs, bis = carry
                    gb = jnp.broadcast_to(gofs + g * 4, (_L,))
                    gx1 = plsc.load_gather(gt_v, [gb])
                    gy1 = plsc.load_gather(gt_v, [gb + 1])
                    gx2 = plsc.load_gather(gt_v, [gb + 2])
                    gy2 = plsc.load_gather(gt_v, [gb + 3])
                    ag = plsc.load_gather(
                        area_v, [jnp.broadcast_to(aofs + g, (_L,))])
                    nbv, nbi = [], []
                    for k in range(_KQ):
                        px1, py1, px2, py2 = _cur[k]
                        w = jnp.maximum(
                            jnp.minimum(px2, gx2) - jnp.maximum(px1, gx1), 0.0)
                        h = jnp.maximum(
                            jnp.minimum(py2, gy2) - jnp.maximum(py1, gy1), 0.0)
                        inter = w * h
                        union = (_areas[k] + ag) - inter
                        iou = inter / union
                        plsc.store_scatter(_slab, [_sbases[k] + g], iou)
                        m = iou > bvs[k]
                        nbv.append(jnp.where(m, iou, bvs[k]))
                        nbi.append(jnp.where(m, g, bis[k]))
                    return tuple(nbv), tuple(nbi)

                bv0 = tuple(jnp.full((_L,), -1.0, jnp.float32)
                            for _ in range(_KQ))
                bi0 = tuple(jnp.zeros((_L,), jnp.int32) for _ in range(_KQ))
                _, bis = lax.fori_loop(0, G, gt_body, (bv0, bi0), unroll=4)

                if s < S - 1:
                    ncur = []
                    for k in range(_KQ):
                        px1, py1, px2, py2 = cur[k]
                        gidx = gofs + bis[k] * 4
                        gx1 = plsc.load_gather(gt_v, [gidx])
                        gy1 = plsc.load_gather(gt_v, [gidx + 1])
                        gx2 = plsc.load_gather(gt_v, [gidx + 2])
                        gy2 = plsc.load_gather(gt_v, [gidx + 3])
                        dcx = (gx1 + gx2) / 2.0 - (px1 + px2) / 2.0
                        dcy = (gy1 + gy2) / 2.0 - (py1 + py2) / 2.0
                        dw = (gx2 - gx1) - (px2 - px1)
                        dh = (gy2 - gy1) - (py2 - py1)
                        ncur.append((
                            px1 + _LR_POS * dcx,
                            py1 + _LR_POS * dcy,
                            ((px2 + _LR_POS * dcx) + _LR_SIZE * dw)
                            - _LR_SIZE * dcx,
                            ((py2 + _LR_POS * dcy) + _LR_SIZE * dh)
                            - _LR_SIZE * dcy,
                        ))
                    cur = ncur

            for s in range(S):
                soff = (b * S + s) * (N * G) + n0q * G
                pltpu.async_copy(
                    slab.at[pl.ds(s * (_KQ * _L * G), _KQ * _L * G)],
                    sims_out.at[pl.ds(soff, _KQ * _L * G)], sem)
                roff = (b * S + s) * (N * 4) + n0q * 4
                pltpu.async_copy(
                    rslab.at[pl.ds(s * (_KQ * _L * 4), _KQ * _L * 4)],
                    res_out.at[pl.ds(roff, _KQ * _L * 4)], sem)

        def pair_body(j2, carry):
            @pl.when(j2 > 0)
            def _():
                drain(slab0, rslab0, sem0)

            process_quad(2 * j2, slab0, rslab0, sem0)

            @pl.when(j2 > 0)
            def _():
                drain(slab1, rslab1, sem1)

            process_quad(2 * j2 + 1, slab1, rslab1, sem1)
            return carry

        lax.fori_loop(0, n_quads // 2, pair_body, 0)
        drain(slab0, rslab0, sem0)
        drain(slab1, rslab1, sem1)

    return pl.kernel(
        body,
        out_type=(
            jax.ShapeDtypeStruct((B * S * N * 4,), jnp.float32),
            jax.ShapeDtypeStruct((B * S * N * G,), jnp.float32),
        ),
        mesh=mesh,
        compiler_params=pltpu.CompilerParams(needs_layout_passes=False),
        scratch_types=[
            pltpu.VMEM((B * G * 4,), jnp.float32),
            pltpu.VMEM((B * G,), jnp.float32),
            pltpu.VMEM((_KQ * _L * 4,), jnp.float32),
            pltpu.VMEM((SLAB,), jnp.float32),
            pltpu.VMEM((SLAB,), jnp.float32),
            pltpu.VMEM((RSLAB,), jnp.float32),
            pltpu.VMEM((RSLAB,), jnp.float32),
            pltpu.SemaphoreType.DMA,
            pltpu.SemaphoreType.DMA,
        ],
    )


@jax.jit
def kernel(boxes, gt_boxes):
    B, N, _ = boxes.shape
    G = gt_boxes.shape[1]
    call = _make_sc_call(B, N, G)
    res, sims = call(boxes.reshape(-1), gt_boxes.reshape(-1))
    return (res.reshape(B, _STAGES, N, 4), sims.reshape(B, _STAGES, N, G))


# speedup vs baseline: 1.0067x; 1.0067x over previous
import functools

import jax
import jax.numpy as jnp
from jax import lax
from jax.experimental import pallas as pl
from jax.experimental.pallas import tpu as pltpu
from jax.experimental.pallas import tpu_sc as plsc

_ITERATIONS = 5
_STAGES = _ITERATIONS + 1
_LR_POS = 0.45
_LR_SIZE = 0.4
_L = 16
_KQ = 4


@functools.lru_cache(maxsize=None)
def _make_sc_call(B, N, G):
    assert B == 2, "kernel specialized to B == 2"
    assert N % _L == 0 and G % _L == 0

    info = plsc.get_sparse_core_info()
    NW = info.num_cores * info.num_subcores
    WPB = NW // B
    nbg = N // _L
    base_loc = nbg // WPB
    n_rem = nbg - base_loc * WPB
    n_quads = -(-(base_loc + 1) // _KQ)
    assert n_quads % 2 == 0, "quad ring expects an even quad count"
    assert base_loc >= _KQ

    S = _STAGES
    GSLAB = S * _L * G
    GRSLAB = S * _L * 4
    SLAB = _KQ * GSLAB
    RSLAB = _KQ * GRSLAB

    mesh = plsc.VectorSubcoreMesh(core_axis_name="c", subcore_axis_name="s")

    def body(boxes_in, gt_in, res_out, sims_out,
             gt_v, area_v, inbox, slab0, slab1, rslab0, rslab1, sem0, sem1):
        wid = lax.axis_index("s") * info.num_cores + lax.axis_index("c")
        b = (wid >= WPB).astype(jnp.int32)
        lid = wid - b * WPB
        loc = base_loc + (lid < n_rem).astype(jnp.int32)
        start = base_loc * lid + jnp.minimum(lid, n_rem)

        pltpu.sync_copy(gt_in, gt_v)

        iota = lax.iota(jnp.int32, _L)
        col4 = iota * 4
        colG = iota * G

        for i in range(B * G // _L):
            ibase = col4 + (i * _L * 4)
            x1 = plsc.load_gather(gt_v, [ibase])
            y1 = plsc.load_gather(gt_v, [ibase + 1])
            x2 = plsc.load_gather(gt_v, [ibase + 2])
            y2 = plsc.load_gather(gt_v, [ibase + 3])
            area_v[pl.ds(i * _L, _L)] = (x2 - x1) * (y2 - y1)

        gofs = b * (G * 4)
        aofs = b * G

        def drain(slab, rslab, sem):
            pltpu.make_async_copy(sims_out.at[pl.ds(0, SLAB)], slab, sem).wait()
            pltpu.make_async_copy(res_out.at[pl.ds(0, RSLAB)], rslab, sem).wait()

        def process_quad(q, slab, rslab, sem):
            jj0 = start + jnp.minimum(q * _KQ, loc - _KQ)
            n0q = jj0 * _L
            cur = []
            pltpu.sync_copy(
                boxes_in.at[pl.ds((b * N + n0q) * 4, _KQ * _L * 4)], inbox)
            for k in range(_KQ):
                kb = col4 + k * (_L * 4)
                cur.append((plsc.load_gather(inbox, [kb]),
                            plsc.load_gather(inbox, [kb + 1]),
                            plsc.load_gather(inbox, [kb + 2]),
                            plsc.load_gather(inbox, [kb + 3])))
            for s in range(S):
                areas, sbases = [], []
                for k in range(_KQ):
                    px1, py1, px2, py2 = cur[k]
                    rbase = col4 + (s * _KQ + k) * (_L * 4)
                    plsc.store_scatter(rslab, [rbase], px1)
                    plsc.store_scatter(rslab, [rbase + 1], py1)
                    plsc.store_scatter(rslab, [rbase + 2], px2)
                    plsc.store_scatter(rslab, [rbase + 3], py2)
                    areas.append((px2 - px1) * (py2 - py1))
                    sbases.append(colG + (s * _KQ + k) * (_L * G))

                def gt_body(g, carry, _cur=tuple(cur), _areas=tuple(areas),
                            _sbases=tuple(sbases), _slab=slab):
                    bvs, bis = carry
                    gb = jnp.broadcast_to(gofs + g * 4, (_L,))
                    gx1 = plsc.load_gather(gt_v, [gb])
                    gy1 = plsc.load_gather(gt_v, [gb + 1])
                    gx2 = plsc.load_gather(gt_v, [gb + 2])
                    gy2 = plsc.load_gather(gt_v, [gb + 3])
                    ag = plsc.load_gather(
                        area_v, [jnp.broadcast_to(aofs + g, (_L,))])
                    nbv, nbi = [], []
                    for k in range(_KQ):
                        px1, py1, px2, py2 = _cur[k]
                        w = jnp.maximum(
                            jnp.minimum(px2, gx2) - jnp.maximum(px1, gx1), 0.0)
                        h = jnp.maximum(
                            jnp.minimum(py2, gy2) - jnp.maximum(py1, gy1), 0.0)
                        inter = w * h
                        union = (_areas[k] + ag) - inter
                        iou = inter / union
                        plsc.store_scatter(_slab, [_sbases[k] + g], iou)
                        m = iou > bvs[k]
                        nbv.append(jnp.where(m, iou, bvs[k]))
                        nbi.append(jnp.where(m, g, bis[k]))
                    return tuple(nbv), tuple(nbi)

                bv0 = tuple(jnp.full((_L,), -1.0, jnp.float32)
                            for _ in range(_KQ))
                bi0 = tuple(jnp.zeros((_L,), jnp.int32) for _ in range(_KQ))
                _, bis = lax.fori_loop(0, G, gt_body, (bv0, bi0), unroll=2)

                if s < S - 1:
                    ncur = []
                    for k in range(_KQ):
                        px1, py1, px2, py2 = cur[k]
                        gidx = gofs + bis[k] * 4
                        gx1 = plsc.load_gather(gt_v, [gidx])
                        gy1 = plsc.load_gather(gt_v, [gidx + 1])
                        gx2 = plsc.load_gather(gt_v, [gidx + 2])
                        gy2 = plsc.load_gather(gt_v, [gidx + 3])
                        dcx = (gx1 + gx2) / 2.0 - (px1 + px2) / 2.0
                        dcy = (gy1 + gy2) / 2.0 - (py1 + py2) / 2.0
                        dw = (gx2 - gx1) - (px2 - px1)
                        dh = (gy2 - gy1) - (py2 - py1)
                        ncur.append((
                            px1 + _LR_POS * dcx,
                            py1 + _LR_POS * dcy,
                            ((px2 + _LR_POS * dcx) + _LR_SIZE * dw)
                            - _LR_SIZE * dcx,
                            ((py2 + _LR_POS * dcy) + _LR_SIZE * dh)
                            - _LR_SIZE * dcy,
                        ))
                    cur = ncur

            for s in range(S):
                soff = (b * S + s) * (N * G) + n0q * G
                pltpu.async_copy(
                    slab.at[pl.ds(s * (_KQ * _L * G), _KQ * _L * G)],
                    sims_out.at[pl.ds(soff, _KQ * _L * G)], sem)
                roff = (b * S + s) * (N * 4) + n0q * 4
                pltpu.async_copy(
                    rslab.at[pl.ds(s * (_KQ * _L * 4), _KQ * _L * 4)],
                    res_out.at[pl.ds(roff, _KQ * _L * 4)], sem)

        def pair_body(j2, carry):
            @pl.when(j2 > 0)
            def _():
                drain(slab0, rslab0, sem0)

            process_quad(2 * j2, slab0, rslab0, sem0)

            @pl.when(j2 > 0)
            def _():
                drain(slab1, rslab1, sem1)

            process_quad(2 * j2 + 1, slab1, rslab1, sem1)
            return carry

        lax.fori_loop(0, n_quads // 2, pair_body, 0)
        drain(slab0, rslab0, sem0)
        drain(slab1, rslab1, sem1)

    return pl.kernel(
        body,
        out_type=(
            jax.ShapeDtypeStruct((B * S * N * 4,), jnp.float32),
            jax.ShapeDtypeStruct((B * S * N * G,), jnp.float32),
        ),
        mesh=mesh,
        compiler_params=pltpu.CompilerParams(needs_layout_passes=False),
        scratch_types=[
            pltpu.VMEM((B * G * 4,), jnp.float32),
            pltpu.VMEM((B * G,), jnp.float32),
            pltpu.VMEM((_KQ * _L * 4,), jnp.float32),
            pltpu.VMEM((SLAB,), jnp.float32),
            pltpu.VMEM((SLAB,), jnp.float32),
            pltpu.VMEM((RSLAB,), jnp.float32),
            pltpu.VMEM((RSLAB,), jnp.float32),
            pltpu.SemaphoreType.DMA,
            pltpu.SemaphoreType.DMA,
        ],
    )


@jax.jit
def kernel(boxes, gt_boxes):
    B, N, _ = boxes.shape
    G = gt_boxes.shape[1]
    call = _make_sc_call(B, N, G)
    res, sims = call(boxes.reshape(-1), gt_boxes.reshape(-1))
    return (res.reshape(B, _STAGES, N, 4), sims.reshape(B, _STAGES, N, G))
